# baseline (speedup 1.0000x reference)
import functools

import jax
import jax.numpy as jnp
from jax import lax
from jax.experimental import pallas as pl
from jax.experimental.pallas import tpu as pltpu
from jax.experimental.pallas import tpu_sc as plsc

_BATCH = 16384
_D = 32
_NC = 2
_NS = 16
_L = 16
_NW = _NC * _NS
_BW = _BATCH // _NW
_CHUNK = 128
_NCHUNK = _BW // _CHUNK
_WD = 0.0001

_mesh = plsc.VectorSubcoreMesh(core_axis_name="c", subcore_axis_name="s")


@functools.partial(
    pl.kernel,
    out_type=(
        jax.ShapeDtypeStruct((_BATCH,), jnp.float32),
        jax.ShapeDtypeStruct((_BATCH,), jnp.float32),
        jax.ShapeDtypeStruct((_NW, _L), jnp.float32),
    ),
    mesh=_mesh,
    compiler_params=pltpu.CompilerParams(
        needs_layout_passes=False, use_tc_tiling_on_sc=False),
    scratch_types=[
        pltpu.VMEM((_BW,), jnp.int32),
        pltpu.VMEM((_BW,), jnp.int32),
        pltpu.VMEM((_BW,), jnp.int32),
        pltpu.VMEM((_BW, _D), jnp.float32),
        pltpu.VMEM((_BW, _D), jnp.float32),
        pltpu.VMEM((_BW, _D), jnp.float32),
        pltpu.VMEM((_BW,), jnp.float32),
        pltpu.VMEM((_BW,), jnp.float32),
        pltpu.VMEM((_L,), jnp.float32),
        pltpu.SemaphoreType.DMA,
    ],
)
def _sc_gather_dot(users, pos, neg, ue, ie,
                   s_pos_out, s_neg_out, sq_out,
                   idx_u, idx_p, idx_n, u_v, p_v, n_v,
                   sp_v, sn_v, sq_v, sem):
    wid = lax.axis_index("s") * _NC + lax.axis_index("c")
    base = wid * _BW

    pltpu.sync_copy(users.at[pl.ds(base, _BW)], idx_u)
    pltpu.sync_copy(pos.at[pl.ds(base, _BW)], idx_p)
    pltpu.sync_copy(neg.at[pl.ds(base, _BW)], idx_n)

    copies = []
    for tbl, idx, dst in ((ue, idx_u, u_v), (ie, idx_p, p_v), (ie, idx_n, n_v)):
        for j in range(_NCHUNK):
            sl = pl.ds(j * _CHUNK, _CHUNK)
            copies.append(pltpu.async_copy(tbl.at[idx.at[sl]], dst.at[sl], sem))
    for c in copies:
        c.wait()

    def body(g, sq_acc):
        rows = lax.iota(jnp.int32, _L) + g * _L
        pos_acc = jnp.zeros((_L,), jnp.float32)
        neg_acc = jnp.zeros((_L,), jnp.float32)
        for d in range(_D):
            cd = jnp.full((_L,), d, jnp.int32)
            u = plsc.load_gather(u_v, [rows, cd])
            p = plsc.load_gather(p_v, [rows, cd])
            n = plsc.load_gather(n_v, [rows, cd])
            pos_acc = pos_acc + u * p
            neg_acc = neg_acc + u * n
            sq_acc = sq_acc + (u * u + p * p + n * n)
        plsc.store_scatter(sp_v, [rows], pos_acc)
        plsc.store_scatter(sn_v, [rows], neg_acc)
        return sq_acc

    sq_acc = lax.fori_loop(0, _BW // _L, body, jnp.zeros((_L,), jnp.float32))
    sq_v[...] = sq_acc

    pltpu.sync_copy(sp_v, s_pos_out.at[pl.ds(base, _BW)])
    pltpu.sync_copy(sn_v, s_neg_out.at[pl.ds(base, _BW)])
    pltpu.sync_copy(sq_v, sq_out.at[wid])


def _tc_loss_body(pos_ref, neg_ref, sq_ref, out_ref):
    sp = pos_ref[...]
    sn = neg_ref[...]
    bce = jnp.sum(jnp.log(1.0 + jnp.exp(-sp))) + jnp.sum(jnp.log(1.0 + jnp.exp(sn)))
    reg = jnp.sum(sq_ref[...])
    out_ref[0, 0] = bce / (2.0 * _BATCH) + _WD * 0.5 * reg / _BATCH


_tc_loss = pl.pallas_call(
    _tc_loss_body,
    out_shape=jax.ShapeDtypeStruct((1, 1), jnp.float32),
    out_specs=pl.BlockSpec(memory_space=pltpu.SMEM),
)


def kernel(users, positive_items, negative_items, user_embedding, item_embedding):
    s_pos, s_neg, sq = _sc_gather_dot(
        users, positive_items, negative_items, user_embedding, item_embedding)
    out = _tc_loss(s_pos.reshape(128, 128), s_neg.reshape(128, 128),
                   sq.reshape(4, 128))
    return out.reshape(())
